# Initial kernel scaffold; baseline (speedup 1.0000x reference)
import functools

import jax
import jax.numpy as jnp
from jax import lax
from jax.experimental import pallas as pl
from jax.experimental.pallas import tpu as pltpu
from jax.experimental.pallas import tpu_sc as plsc

N = 320000
NSEG = 10000
D = 128
BN_EPS = 1e-5

NC = 2
NS = 16
NW = NC * NS
ROWS_PER_W = N // NW
CH = 125
NCH = ROWS_PER_W // CH
SEG_SLICE = NSEG // NS

_mesh = plsc.VectorSubcoreMesh(core_axis_name="c", subcore_axis_name="s")


@functools.partial(
    pl.kernel,
    out_type=jax.ShapeDtypeStruct((NC, NSEG, D), jnp.float32),
    mesh=_mesh,
    scratch_types=[
        pltpu.VMEM((NCH, CH), jnp.int32),
        pltpu.VMEM((CH, D), jnp.float32),
        pltpu.VMEM_SHARED((NSEG, D), jnp.float32),
    ],
)
def _segment_sum_sc(x_hbm, batch_hbm, zeros_hbm, out_hbm, idx_v, xbuf, table_sh):
    c = lax.axis_index("c")
    s = lax.axis_index("s")
    wid = c * NS + s
    base = wid * ROWS_PER_W

    pltpu.sync_copy(batch_hbm.at[wid], idx_v)
    pltpu.sync_copy(zeros_hbm, table_sh.at[pl.ds(s * SEG_SLICE, SEG_SLICE)])
    plsc.subcore_barrier()

    def body(j, carry):
        pltpu.sync_copy(x_hbm.at[pl.ds(base + j * CH, CH)], xbuf)
        pltpu.sync_copy(xbuf, table_sh.at[idx_v.at[j]], add=True)
        return carry

    lax.fori_loop(0, NCH, body, 0)
    plsc.subcore_barrier()
    pltpu.sync_copy(
        table_sh.at[pl.ds(s * SEG_SLICE, SEG_SLICE)],
        out_hbm.at[c, pl.ds(s * SEG_SLICE, SEG_SLICE)],
    )


def _bn_body(p_ref, wlin_ref, gamma_ref, beta_ref, out_ref):
    summ = p_ref[0] + p_ref[1]
    summ = lax.dot_general(
        summ, wlin_ref[...], (((1,), (1,)), ((), ())),
        preferred_element_type=jnp.float32,
    )
    mean = jnp.mean(summ, axis=0, keepdims=True)
    var = jnp.mean((summ - mean) ** 2, axis=0, keepdims=True)
    y = (summ - mean) / jnp.sqrt(var + BN_EPS) * gamma_ref[...] + beta_ref[...]
    out_ref[...] = jnp.maximum(y, 0.0)


_bn_call = pl.pallas_call(
    _bn_body,
    out_shape=jax.ShapeDtypeStruct((NSEG, D), jnp.float32),
)


@functools.partial(
    pl.kernel,
    out_type=jax.ShapeDtypeStruct((N, D), jnp.float32),
    mesh=_mesh,
    scratch_types=[
        pltpu.VMEM((NCH, CH), jnp.int32),
        pltpu.VMEM((CH, D), jnp.float32),
        pltpu.SemaphoreType.DMA,
    ],
)
def _gather_sc(table_hbm, batch_hbm, out_hbm, idx_v, rows_v, sem):
    c = lax.axis_index("c")
    s = lax.axis_index("s")
    wid = c * NS + s
    base = wid * ROWS_PER_W

    pltpu.sync_copy(batch_hbm.at[wid], idx_v)

    def body(j, carry):
        pltpu.async_copy(table_hbm.at[idx_v.at[j]], rows_v, sem).wait()
        pltpu.sync_copy(rows_v, out_hbm.at[pl.ds(base + j * CH, CH)])
        return carry

    lax.fori_loop(0, NCH, body, 0)


_BR = 512


def _mlp_body(x_ref, g_ref, w1_ref, w2_ref, out_ref):
    h = x_ref[...] + g_ref[...]
    h = lax.dot_general(
        h, w1_ref[...], (((1,), (1,)), ((), ())),
        preferred_element_type=jnp.float32,
    )
    h = jnp.maximum(h, 0.0)
    out_ref[...] = lax.dot_general(
        h, w2_ref[...], (((1,), (1,)), ((), ())),
        preferred_element_type=jnp.float32,
    )


_mlp_call = pl.pallas_call(
    _mlp_body,
    grid=(N // _BR,),
    in_specs=[
        pl.BlockSpec((_BR, D), lambda i: (i, 0)),
        pl.BlockSpec((_BR, D), lambda i: (i, 0)),
        pl.BlockSpec((D, D), lambda i: (0, 0)),
        pl.BlockSpec((D, D), lambda i: (0, 0)),
    ],
    out_specs=pl.BlockSpec((_BR, D), lambda i: (i, 0)),
    out_shape=jax.ShapeDtypeStruct((N, D), jnp.float32),
)


def kernel(x, edge_index, edge_attr, batch, W_lin, gamma, beta, W1, W2):
    del edge_index, edge_attr
    batch3 = batch.reshape(NW, NCH, CH)
    zeros = jnp.zeros((SEG_SLICE, D), jnp.float32)
    partials = _segment_sum_sc(x, batch3, zeros)
    table = _bn_call(partials, W_lin, gamma, beta)
    g = _gather_sc(table, batch3)
    return _mlp_call(x, g, W1, W2)


# speedup vs baseline: 1.5132x; 1.5132x over previous
import functools

import jax
import jax.numpy as jnp
from jax import lax
from jax.experimental import pallas as pl
from jax.experimental.pallas import tpu as pltpu
from jax.experimental.pallas import tpu_sc as plsc

N = 320000
NSEG = 10000
NSEG_PAD = 10240
D = 128
BN_EPS = 1e-5

NC = 2
NS = 16
NW = NC * NS
ROWS_PER_W = N // NW
CH = 80
NCH = ROWS_PER_W // CH
SEG_SLICE = NSEG_PAD // NS

_mesh = plsc.VectorSubcoreMesh(core_axis_name="c", subcore_axis_name="s")


@functools.partial(
    pl.kernel,
    out_type=jax.ShapeDtypeStruct((NC, NSEG_PAD, D), jnp.float32),
    mesh=_mesh,
    scratch_types=[
        pltpu.VMEM((NCH, CH), jnp.int32),
        pltpu.VMEM((CH, D), jnp.float32),
        pltpu.VMEM_SHARED((NSEG_PAD, D), jnp.float32),
    ],
)
def _segment_sum_sc(x_hbm, batch_hbm, zeros_hbm, out_hbm, idx_v, xbuf, table_sh):
    c = lax.axis_index("c")
    s = lax.axis_index("s")
    wid = c * NS + s
    base = wid * ROWS_PER_W

    pltpu.sync_copy(batch_hbm.at[wid], idx_v)
    pltpu.sync_copy(zeros_hbm, table_sh.at[pl.ds(s * SEG_SLICE, SEG_SLICE)])
    plsc.subcore_barrier()

    def body(j, carry):
        pltpu.sync_copy(x_hbm.at[pl.ds(base + j * CH, CH)], xbuf)
        pltpu.sync_copy(xbuf, table_sh.at[idx_v.at[j]], add=True)
        return carry

    lax.fori_loop(0, NCH, body, 0)
    plsc.subcore_barrier()
    pltpu.sync_copy(
        table_sh.at[pl.ds(s * SEG_SLICE, SEG_SLICE)],
        out_hbm.at[c, pl.ds(s * SEG_SLICE, SEG_SLICE)],
    )


def _bn_body(p_ref, wlin_ref, gamma_ref, beta_ref, out_ref):
    summ = p_ref[0] + p_ref[1]
    summ = lax.dot_general(
        summ, wlin_ref[...], (((1,), (1,)), ((), ())),
        preferred_element_type=jnp.float32,
    )
    mean = jnp.sum(summ, axis=0, keepdims=True) / NSEG
    cent = summ - mean
    ssq = jnp.sum(cent * cent, axis=0, keepdims=True) - (
        (NSEG_PAD - NSEG) * mean * mean
    )
    var = ssq / NSEG
    y = cent / jnp.sqrt(var + BN_EPS) * gamma_ref[...] + beta_ref[...]
    out_ref[...] = jnp.maximum(y, 0.0)


_bn_call = pl.pallas_call(
    _bn_body,
    out_shape=jax.ShapeDtypeStruct((NSEG_PAD, D), jnp.float32),
)


@functools.partial(
    pl.kernel,
    out_type=jax.ShapeDtypeStruct((N, D), jnp.float32),
    mesh=_mesh,
    scratch_types=[
        pltpu.VMEM((NCH, CH), jnp.int32),
        pltpu.VMEM((CH, D), jnp.float32),
        pltpu.SemaphoreType.DMA,
    ],
)
def _gather_sc(table_hbm, batch_hbm, out_hbm, idx_v, rows_v, sem):
    c = lax.axis_index("c")
    s = lax.axis_index("s")
    wid = c * NS + s
    base = wid * ROWS_PER_W

    pltpu.sync_copy(batch_hbm.at[wid], idx_v)

    def body(j, carry):
        pltpu.async_copy(table_hbm.at[idx_v.at[j]], rows_v, sem).wait()
        pltpu.sync_copy(rows_v, out_hbm.at[pl.ds(base + j * CH, CH)])
        return carry

    lax.fori_loop(0, NCH, body, 0)


_BR = 512


def _mlp_body(x_ref, g_ref, w1_ref, w2_ref, out_ref):
    h = x_ref[...] + g_ref[...]
    h = lax.dot_general(
        h, w1_ref[...], (((1,), (1,)), ((), ())),
        preferred_element_type=jnp.float32,
    )
    h = jnp.maximum(h, 0.0)
    out_ref[...] = lax.dot_general(
        h, w2_ref[...], (((1,), (1,)), ((), ())),
        preferred_element_type=jnp.float32,
    )


_mlp_call = pl.pallas_call(
    _mlp_body,
    grid=(N // _BR,),
    in_specs=[
        pl.BlockSpec((_BR, D), lambda i: (i, 0)),
        pl.BlockSpec((_BR, D), lambda i: (i, 0)),
        pl.BlockSpec((D, D), lambda i: (0, 0)),
        pl.BlockSpec((D, D), lambda i: (0, 0)),
    ],
    out_specs=pl.BlockSpec((_BR, D), lambda i: (i, 0)),
    out_shape=jax.ShapeDtypeStruct((N, D), jnp.float32),
)


def kernel(x, edge_index, edge_attr, batch, W_lin, gamma, beta, W1, W2):
    del edge_index, edge_attr
    batch3 = batch.reshape(NW, NCH, CH)
    zeros = jnp.zeros((SEG_SLICE, D), jnp.float32)
    partials = _segment_sum_sc(x, batch3, zeros)
    table = _bn_call(partials, W_lin, gamma.reshape(1, D), beta.reshape(1, D))
    g = _gather_sc(table, batch3)
    return _mlp_call(x, g, W1, W2)


# speedup vs baseline: 2.1167x; 1.3988x over previous
import functools

import jax
import jax.numpy as jnp
from jax import lax
from jax.experimental import pallas as pl
from jax.experimental.pallas import tpu as pltpu
from jax.experimental.pallas import tpu_sc as plsc

N = 320000
NSEG = 10000
NSEG_PAD = 10240
D = 128
BN_EPS = 1e-5

NC = 2
NS = 16
NW = NC * NS
ROWS_PER_W = N // NW
CH = 80
NCH = ROWS_PER_W // CH
SEG_SLICE = NSEG_PAD // NS

_mesh = plsc.VectorSubcoreMesh(core_axis_name="c", subcore_axis_name="s")


@functools.partial(
    pl.kernel,
    out_type=jax.ShapeDtypeStruct((NC, NSEG_PAD, D), jnp.float32),
    mesh=_mesh,
    scratch_types=[
        pltpu.VMEM((NCH, CH), jnp.int32),
        pltpu.VMEM((CH, D), jnp.float32),
        pltpu.VMEM((CH, D), jnp.float32),
        pltpu.VMEM_SHARED((NSEG_PAD, D), jnp.float32),
        pltpu.SemaphoreType.DMA,
        pltpu.SemaphoreType.DMA,
        pltpu.SemaphoreType.DMA,
        pltpu.SemaphoreType.DMA,
    ],
)
def _segment_sum_sc(x_hbm, batch_hbm, zeros_hbm, out_hbm,
                    idx_v, xa, xb, table_sh, gsa, gsb, ssa, ssb):
    c = lax.axis_index("c")
    s = lax.axis_index("s")
    wid = c * NS + s
    base = wid * ROWS_PER_W

    def chunk(j):
        return x_hbm.at[pl.ds(base + j * CH, CH)]

    pltpu.sync_copy(batch_hbm.at[wid], idx_v)
    pltpu.sync_copy(zeros_hbm, table_sh.at[pl.ds(s * SEG_SLICE, SEG_SLICE)])
    plsc.subcore_barrier()

    pltpu.async_copy(chunk(0), xa, gsa)

    def body(j2, carry):
        j = 2 * j2
        pltpu.make_async_copy(chunk(j), xa, gsa).wait()

        @pl.when(j2 > 0)
        def _():
            pltpu.make_async_copy(xb, table_sh.at[idx_v.at[j]], ssb).wait()

        pltpu.async_copy(chunk(j + 1), xb, gsb)
        sca = pltpu.async_copy(xa, table_sh.at[idx_v.at[j]], ssa, add=True)
        pltpu.make_async_copy(chunk(j + 1), xb, gsb).wait()
        sca.wait()
        pltpu.async_copy(chunk(j + 2), xa, gsa)
        pltpu.async_copy(xb, table_sh.at[idx_v.at[j + 1]], ssb, add=True)
        return carry

    lax.fori_loop(0, (NCH - 1) // 2, body, 0)
    pltpu.make_async_copy(chunk(NCH - 1), xa, gsa).wait()
    pltpu.make_async_copy(xb, table_sh.at[idx_v.at[NCH - 2]], ssb).wait()
    pltpu.sync_copy(xa, table_sh.at[idx_v.at[NCH - 1]], add=True)
    plsc.subcore_barrier()
    pltpu.sync_copy(
        table_sh.at[pl.ds(s * SEG_SLICE, SEG_SLICE)],
        out_hbm.at[c, pl.ds(s * SEG_SLICE, SEG_SLICE)],
    )


def _bn_body(p_ref, wlin_ref, gamma_ref, beta_ref, out_ref):
    summ = p_ref[0] + p_ref[1]
    summ = lax.dot_general(
        summ, wlin_ref[...], (((1,), (1,)), ((), ())),
        preferred_element_type=jnp.float32,
    )
    mean = jnp.sum(summ, axis=0, keepdims=True) / NSEG
    cent = summ - mean
    ssq = jnp.sum(cent * cent, axis=0, keepdims=True) - (
        (NSEG_PAD - NSEG) * mean * mean
    )
    var = ssq / NSEG
    y = cent / jnp.sqrt(var + BN_EPS) * gamma_ref[...] + beta_ref[...]
    out_ref[...] = jnp.maximum(y, 0.0)


_bn_call = pl.pallas_call(
    _bn_body,
    out_shape=jax.ShapeDtypeStruct((NSEG_PAD, D), jnp.float32),
)


@functools.partial(
    pl.kernel,
    out_type=jax.ShapeDtypeStruct((N, D), jnp.float32),
    mesh=_mesh,
    scratch_types=[
        pltpu.VMEM((NCH, CH), jnp.int32),
        pltpu.VMEM((CH, D), jnp.float32),
        pltpu.VMEM((CH, D), jnp.float32),
        pltpu.SemaphoreType.DMA,
        pltpu.SemaphoreType.DMA,
        pltpu.SemaphoreType.DMA,
        pltpu.SemaphoreType.DMA,
    ],
)
def _gather_sc(table_hbm, batch_hbm, out_hbm, idx_v, ga, gb, gsa, gsb, wsa, wsb):
    c = lax.axis_index("c")
    s = lax.axis_index("s")
    wid = c * NS + s
    base = wid * ROWS_PER_W

    def outref(j):
        return out_hbm.at[pl.ds(base + j * CH, CH)]

    pltpu.sync_copy(batch_hbm.at[wid], idx_v)
    pltpu.async_copy(table_hbm.at[idx_v.at[0]], ga, gsa)

    def body(j2, carry):
        j = 2 * j2
        pltpu.make_async_copy(table_hbm.at[idx_v.at[j]], ga, gsa).wait()

        @pl.when(j2 > 0)
        def _():
            pltpu.make_async_copy(gb, outref(j - 1), wsb).wait()

        pltpu.async_copy(table_hbm.at[idx_v.at[j + 1]], gb, gsb)
        wa = pltpu.async_copy(ga, outref(j), wsa)
        pltpu.make_async_copy(table_hbm.at[idx_v.at[j + 1]], gb, gsb).wait()
        wa.wait()
        pltpu.async_copy(table_hbm.at[idx_v.at[j + 2]], ga, gsa)
        pltpu.async_copy(gb, outref(j + 1), wsb)
        return carry

    lax.fori_loop(0, (NCH - 1) // 2, body, 0)
    pltpu.make_async_copy(table_hbm.at[idx_v.at[NCH - 1]], ga, gsa).wait()
    pltpu.make_async_copy(gb, outref(NCH - 2), wsb).wait()
    pltpu.sync_copy(ga, outref(NCH - 1))


_BR = 2000


def _mlp_body(x_ref, g_ref, w1_ref, w2_ref, out_ref):
    h = x_ref[...] + g_ref[...]
    h = lax.dot_general(
        h, w1_ref[...], (((1,), (1,)), ((), ())),
        preferred_element_type=jnp.float32,
    )
    h = jnp.maximum(h, 0.0)
    out_ref[...] = lax.dot_general(
        h, w2_ref[...], (((1,), (1,)), ((), ())),
        preferred_element_type=jnp.float32,
    )


_mlp_call = pl.pallas_call(
    _mlp_body,
    grid=(N // _BR,),
    in_specs=[
        pl.BlockSpec((_BR, D), lambda i: (i, 0)),
        pl.BlockSpec((_BR, D), lambda i: (i, 0)),
        pl.BlockSpec((D, D), lambda i: (0, 0)),
        pl.BlockSpec((D, D), lambda i: (0, 0)),
    ],
    out_specs=pl.BlockSpec((_BR, D), lambda i: (i, 0)),
    out_shape=jax.ShapeDtypeStruct((N, D), jnp.float32),
)


def kernel(x, edge_index, edge_attr, batch, W_lin, gamma, beta, W1, W2):
    del edge_index, edge_attr
    batch3 = batch.reshape(NW, NCH, CH)
    zeros = jnp.zeros((SEG_SLICE, D), jnp.float32)
    partials = _segment_sum_sc(x, batch3, zeros)
    table = _bn_call(partials, W_lin, gamma.reshape(1, D), beta.reshape(1, D))
    g = _gather_sc(table, batch3)
    return _mlp_call(x, g, W1, W2)


# speedup vs baseline: 3.9149x; 1.8495x over previous
import functools

import jax
import jax.numpy as jnp
from jax import lax
from jax.experimental import pallas as pl
from jax.experimental.pallas import tpu as pltpu
from jax.experimental.pallas import tpu_sc as plsc

N = 320000
NSEG = 10000
NSEG_PAD = 10240
D = 128
BN_EPS = 1e-5

NC = 2
NS = 16
NW = NC * NS
ROWS_PER_W = N // NW
CH = 80
NCH = ROWS_PER_W // CH
SEG_SLICE = NSEG_PAD // NS

_mesh = plsc.VectorSubcoreMesh(core_axis_name="c", subcore_axis_name="s")


@functools.partial(
    pl.kernel,
    out_type=jax.ShapeDtypeStruct((NC, NSEG_PAD, D), jnp.float32),
    mesh=_mesh,
    scratch_types=[
        pltpu.VMEM((NCH, CH), jnp.int32),
        pltpu.VMEM((CH, D), jnp.float32),
        pltpu.VMEM((CH, D), jnp.float32),
        pltpu.VMEM_SHARED((NSEG_PAD, D), jnp.float32),
        pltpu.SemaphoreType.DMA,
        pltpu.SemaphoreType.DMA,
        pltpu.SemaphoreType.DMA,
        pltpu.SemaphoreType.DMA,
    ],
)
def _segment_sum_sc(x_hbm, batch_hbm, zeros_hbm, out_hbm,
                    idx_v, xa, xb, table_sh, gsa, gsb, ssa, ssb):
    c = lax.axis_index("c")
    s = lax.axis_index("s")
    wid = c * NS + s
    base = wid * ROWS_PER_W

    def chunk(j):
        return x_hbm.at[pl.ds(base + j * CH, CH)]

    pltpu.sync_copy(batch_hbm.at[wid], idx_v)
    pltpu.sync_copy(zeros_hbm, table_sh.at[pl.ds(s * SEG_SLICE, SEG_SLICE)])
    plsc.subcore_barrier()

    pltpu.async_copy(chunk(0), xa, gsa)

    def body(j2, carry):
        j = 2 * j2
        pltpu.make_async_copy(chunk(j), xa, gsa).wait()

        @pl.when(j2 > 0)
        def _():
            pltpu.make_async_copy(xb, table_sh.at[idx_v.at[j]], ssb).wait()

        pltpu.async_copy(chunk(j + 1), xb, gsb)
        sca = pltpu.async_copy(xa, table_sh.at[idx_v.at[j]], ssa, add=True)
        pltpu.make_async_copy(chunk(j + 1), xb, gsb).wait()
        sca.wait()
        pltpu.async_copy(chunk(j + 2), xa, gsa)
        pltpu.async_copy(xb, table_sh.at[idx_v.at[j + 1]], ssb, add=True)
        return carry

    lax.fori_loop(0, (NCH - 1) // 2, body, 0)
    pltpu.make_async_copy(chunk(NCH - 1), xa, gsa).wait()
    pltpu.make_async_copy(xb, table_sh.at[idx_v.at[NCH - 2]], ssb).wait()
    pltpu.sync_copy(xa, table_sh.at[idx_v.at[NCH - 1]], add=True)
    plsc.subcore_barrier()
    pltpu.sync_copy(
        table_sh.at[pl.ds(s * SEG_SLICE, SEG_SLICE)],
        out_hbm.at[c, pl.ds(s * SEG_SLICE, SEG_SLICE)],
    )


def _bn_body(p_ref, wlin_ref, gamma_ref, beta_ref, out_ref):
    summ = p_ref[0] + p_ref[1]
    summ = lax.dot_general(
        summ, wlin_ref[...], (((1,), (1,)), ((), ())),
        preferred_element_type=jnp.float32,
    )
    mean = jnp.sum(summ, axis=0, keepdims=True) / NSEG
    cent = summ - mean
    ssq = jnp.sum(cent * cent, axis=0, keepdims=True) - (
        (NSEG_PAD - NSEG) * mean * mean
    )
    var = ssq / NSEG
    y = cent / jnp.sqrt(var + BN_EPS) * gamma_ref[...] + beta_ref[...]
    out_ref[...] = jnp.maximum(y, 0.0)


_bn_call = pl.pallas_call(
    _bn_body,
    out_shape=jax.ShapeDtypeStruct((NSEG_PAD, D), jnp.float32),
)


@functools.partial(
    pl.kernel,
    out_type=jax.ShapeDtypeStruct((N, D), jnp.float32),
    mesh=_mesh,
    scratch_types=[
        pltpu.VMEM((NCH, CH), jnp.int32),
        pltpu.VMEM((CH, D), jnp.float32),
        pltpu.VMEM((CH, D), jnp.float32),
        pltpu.VMEM_SHARED((NSEG_PAD, D), jnp.float32),
        pltpu.SemaphoreType.DMA,
        pltpu.SemaphoreType.DMA,
        pltpu.SemaphoreType.DMA,
        pltpu.SemaphoreType.DMA,
    ],
)
def _gather_sc(table_hbm, batch_hbm, out_hbm,
               idx_v, ga, gb, table_sh, gsa, gsb, wsa, wsb):
    c = lax.axis_index("c")
    s = lax.axis_index("s")
    wid = c * NS + s
    base = wid * ROWS_PER_W

    def outref(j):
        return out_hbm.at[pl.ds(base + j * CH, CH)]

    pltpu.sync_copy(
        table_hbm.at[pl.ds(s * SEG_SLICE, SEG_SLICE)],
        table_sh.at[pl.ds(s * SEG_SLICE, SEG_SLICE)],
    )
    pltpu.sync_copy(batch_hbm.at[wid], idx_v)
    plsc.subcore_barrier()

    pltpu.async_copy(table_sh.at[idx_v.at[0]], ga, gsa)

    def body(j2, carry):
        j = 2 * j2
        pltpu.make_async_copy(table_sh.at[idx_v.at[j]], ga, gsa).wait()

        @pl.when(j2 > 0)
        def _():
            pltpu.make_async_copy(gb, outref(j - 1), wsb).wait()

        pltpu.async_copy(table_sh.at[idx_v.at[j + 1]], gb, gsb)
        wa = pltpu.async_copy(ga, outref(j), wsa)
        pltpu.make_async_copy(table_sh.at[idx_v.at[j + 1]], gb, gsb).wait()
        wa.wait()
        pltpu.async_copy(table_sh.at[idx_v.at[j + 2]], ga, gsa)
        pltpu.async_copy(gb, outref(j + 1), wsb)
        return carry

    lax.fori_loop(0, (NCH - 1) // 2, body, 0)
    pltpu.make_async_copy(table_sh.at[idx_v.at[NCH - 1]], ga, gsa).wait()
    pltpu.make_async_copy(gb, outref(NCH - 2), wsb).wait()
    pltpu.sync_copy(ga, outref(NCH - 1))


_BR = 2000


def _mlp_body(x_ref, g_ref, w1_ref, w2_ref, out_ref):
    h = x_ref[...] + g_ref[...]
    h = lax.dot_general(
        h, w1_ref[...], (((1,), (1,)), ((), ())),
        preferred_element_type=jnp.float32,
    )
    h = jnp.maximum(h, 0.0)
    out_ref[...] = lax.dot_general(
        h, w2_ref[...], (((1,), (1,)), ((), ())),
        preferred_element_type=jnp.float32,
    )


_mlp_call = pl.pallas_call(
    _mlp_body,
    grid=(N // _BR,),
    in_specs=[
        pl.BlockSpec((_BR, D), lambda i: (i, 0)),
        pl.BlockSpec((_BR, D), lambda i: (i, 0)),
        pl.BlockSpec((D, D), lambda i: (0, 0)),
        pl.BlockSpec((D, D), lambda i: (0, 0)),
    ],
    out_specs=pl.BlockSpec((_BR, D), lambda i: (i, 0)),
    out_shape=jax.ShapeDtypeStruct((N, D), jnp.float32),
)


def kernel(x, edge_index, edge_attr, batch, W_lin, gamma, beta, W1, W2):
    del edge_index, edge_attr
    batch3 = batch.reshape(NW, NCH, CH)
    zeros = jnp.zeros((SEG_SLICE, D), jnp.float32)
    partials = _segment_sum_sc(x, batch3, zeros)
    table = _bn_call(partials, W_lin, gamma.reshape(1, D), beta.reshape(1, D))
    g = _gather_sc(table, batch3)
    return _mlp_call(x, g, W1, W2)
